# baseline (speedup 1.0000x reference)
import numpy as np
import jax
import jax.numpy as jnp
from jax.experimental import pallas as pl
from jax.experimental.pallas import tpu as pltpu

_NEG = -1e30
_H = 32
_W = 32
_C = 128
_SP = _W + 3
_FLAT = 1360
_INT = 112
_M = _H * _SP
_CLEN = 1344


def _ring_rel():
    c = 3
    taps = [(dy, dx) for dy in range(7) for dx in range(7)]
    taps.sort(key=lambda t: (max(abs(t[0] - c), abs(t[1] - c)), t[0], t[1]))
    return [(dy - 3) * _SP + (dx - 3) for dy, dx in taps]


_REL = _ring_rel()

_ZA = [(0, 1), (2, 3), (4, 5), (6, 7), (8, None)]
_ZB = [(9 + 2 * k, 10 + 2 * k) for k in range(8)]
_ZC = [(25 + 2 * k, 26 + 2 * k) for k in range(12)]


def _interior_mask():
    f = np.arange(_FLAT)
    i, j = (f - _INT) // _SP, (f - _INT) % _SP
    m = (f >= _INT) & (i < _H) & (j < _W)
    return np.repeat(m[:, None].astype(np.float32), _C, axis=1)


_MASK = _interior_mask()
_MKEEP = _MASK[_INT:_INT + _M, :].astype(np.float32)
_MFILL = ((1.0 - _MKEEP) * _NEG).astype(np.float32)


def _module_kernel(x_ref, mask_ref,
                   wbn0_ref, sbn0_ref, w10_ref, s10_ref,
                   wa0_ref, wb0_ref, wc0_ref, s30_ref, s50_ref, s70_ref,
                   wbn1_ref, sbn1_ref, w11_ref, s11_ref,
                   wa1_ref, wb1_ref, wc1_ref, s31_ref, s51_ref, s71_ref,
                   mkeep_ref, mfill_ref,
                   o_ref, xflat, bnflat, pbuf):
    f32 = jnp.float32
    bf16 = jnp.bfloat16
    mask = mask_ref[...]

    xflat[...] = jnp.full((_FLAT, _C), _NEG, bf16)

    def tap(t):
        a = _INT + _REL[t]
        return bnflat[a:a + _M, :]

    def ptap(i, j):
        a = tap(i)
        if j is None:
            return a
        b = tap(j)
        return a + jnp.concatenate([b[:, 64:128], b[:, 0:64]], axis=1)

    def one_layer(vwin, wbn_ref, sbn_ref, w1_ref, s1_ref,
                  wa_ref, wb_ref, wc_ref, s3_ref, s5_ref, s7_ref):
        xflat[_INT:_INT + _M, :] = (vwin * mkeep_ref[...]
                                    + mfill_ref[...])

        g0 = _INT - 2 * _SP - 2
        cl = 1260
        cm3 = jnp.maximum(
            jnp.maximum(xflat[g0 + 1:g0 + 1 + cl, :],
                        xflat[g0 + 2:g0 + 2 + cl, :]),
            xflat[g0 + 3:g0 + 3 + cl, :])
        cm5 = jnp.maximum(jnp.maximum(cm3, xflat[g0:g0 + cl, :]),
                          xflat[g0 + 4:g0 + 4 + cl, :])
        p3 = jnp.maximum(
            jnp.maximum(cm3[_SP:_SP + _M], cm3[2 * _SP:2 * _SP + _M]),
            cm3[3 * _SP:3 * _SP + _M])
        p5 = jnp.maximum(
            jnp.maximum(jnp.maximum(cm5[0:_M], cm5[_SP:_SP + _M]),
                        jnp.maximum(cm5[2 * _SP:2 * _SP + _M],
                                    cm5[3 * _SP:3 * _SP + _M])),
            cm5[4 * _SP:4 * _SP + _M])
        pbuf[...] = p3 + p5

        sbn = sbn_ref[...]
        bn = jnp.dot(xflat[...], wbn_ref[...], preferred_element_type=f32)
        bn = jnp.maximum(bn * sbn[0:1, :] + sbn[1:2, :], 0.0) * mask
        bnflat[...] = bn.astype(bf16)

        s1 = s1_ref[...]
        cv1 = jnp.dot(tap(0), w1_ref[...], preferred_element_type=f32)
        o1 = jnp.maximum(cv1 * s1[0:1, :] + s1[1:2, :], 0.0)

        la = jnp.concatenate([ptap(i, j) for i, j in _ZA], axis=1)
        da = jnp.dot(la, wa_ref[...], preferred_element_type=f32)
        lb = jnp.concatenate([ptap(i, j) for i, j in _ZB], axis=1)
        db = jnp.dot(lb, wb_ref[...], preferred_element_type=f32)
        lc = jnp.concatenate([ptap(i, j) for i, j in _ZC], axis=1)
        dc = jnp.dot(lc, wc_ref[...], preferred_element_type=f32)

        s3 = s3_ref[...]
        s5 = s5_ref[...]
        s7 = s7_ref[...]
        o3 = jnp.maximum(da[:, 0:_C] * s3[0:1, :] + s3[1:2, :], 0.0)
        c5 = da[:, _C:2 * _C] + db[:, 0:_C]
        o5 = jnp.maximum(c5 * s5[0:1, :] + s5[1:2, :], 0.0)
        c7 = da[:, 2 * _C:3 * _C] + db[:, _C:2 * _C] + dc
        o7 = jnp.maximum(c7 * s7[0:1, :] + s7[1:2, :], 0.0)

        return o1 + o3 + o5 + o7 + pbuf[...].astype(f32)

    x0 = x_ref[0]
    v0 = jnp.concatenate(
        [x0.astype(bf16), jnp.zeros((_H, _SP - _W, _C), bf16)],
        axis=1).reshape(_M, _C)
    r0 = one_layer(v0, wbn0_ref, sbn0_ref, w10_ref, s10_ref,
                   wa0_ref, wb0_ref, wc0_ref, s30_ref, s50_ref, s70_ref)
    r1 = one_layer(r0.astype(bf16), wbn1_ref, sbn1_ref, w11_ref, s11_ref,
                   wa1_ref, wb1_ref, wc1_ref, s31_ref, s51_ref, s71_ref)
    o_ref[0] = r1.reshape(_H, _SP, _C)[:, 0:_W, :]


def _pack_rows(W, pairs):
    rows = []
    for i, j in pairs:
        rows.append(W[i * _C:i * _C + 64])
        rows.append(W[j * _C:j * _C + 64] if j is not None
                    else jnp.zeros((64, W.shape[1]), W.dtype))
    return jnp.concatenate(rows, axis=0)


def _prep_layer(wbn, sbn, w1, s1, w3, s3, w5, s5, w7, s7):
    bf16 = jnp.bfloat16
    k3 = w3.shape[0]
    wa = _pack_rows(jnp.concatenate([w3, w5[0:k3], w7[0:k3]], axis=1),
                    _ZA).astype(bf16)
    wb = _pack_rows(jnp.concatenate([w5, w7[0:w5.shape[0]]], axis=1),
                    _ZB).astype(bf16)
    wc = _pack_rows(w7, _ZC).astype(bf16)
    return (wbn.astype(bf16), sbn, w1.astype(bf16), s1,
            wa, wb, wc, s3, s5, s7)


def _const_spec(a):
    return pl.BlockSpec(a.shape, lambda b: (0, 0))


def kernel(x,
           L0_wbn, L0_sbn, L0_w1, L0_s1, L0_w3, L0_s3, L0_w5, L0_s5,
           L0_w7, L0_s7,
           L1_wbn, L1_sbn, L1_w1, L1_s1, L1_w3, L1_s3, L1_w5, L1_s5,
           L1_w7, L1_s7):
    N = x.shape[0]
    xh = jnp.transpose(x, (0, 2, 3, 1))
    consts = ((jnp.asarray(_MASK),)
              + _prep_layer(L0_wbn, L0_sbn, L0_w1, L0_s1, L0_w3, L0_s3,
                            L0_w5, L0_s5, L0_w7, L0_s7)
              + _prep_layer(L1_wbn, L1_sbn, L1_w1, L1_s1, L1_w3, L1_s3,
                            L1_w5, L1_s5, L1_w7, L1_s7)
              + (jnp.asarray(_MKEEP).astype(jnp.bfloat16),
                 jnp.asarray(_MFILL).astype(jnp.bfloat16)))
    in_specs = [pl.BlockSpec((1, _H, _W, _C), lambda b: (b, 0, 0, 0))]
    in_specs += [_const_spec(a) for a in consts]
    y = pl.pallas_call(
        _module_kernel,
        out_shape=jax.ShapeDtypeStruct((N, _H, _W, _C), jnp.float32),
        grid_spec=pltpu.PrefetchScalarGridSpec(
            num_scalar_prefetch=0, grid=(N,),
            in_specs=in_specs,
            out_specs=pl.BlockSpec((1, _H, _W, _C), lambda b: (b, 0, 0, 0)),
            scratch_shapes=[pltpu.VMEM((_FLAT, _C), jnp.bfloat16),
                            pltpu.VMEM((_FLAT, _C), jnp.bfloat16),
                            pltpu.VMEM((_M, _C), jnp.bfloat16)]),
        compiler_params=pltpu.CompilerParams(
            dimension_semantics=("parallel",),
            vmem_limit_bytes=60 * 1024 * 1024),
    )(xh, *consts)
    return jnp.transpose(y, (0, 3, 1, 2))


# speedup vs baseline: 1.0997x; 1.0997x over previous
import numpy as np
import jax
import jax.numpy as jnp
from jax.experimental import pallas as pl
from jax.experimental.pallas import tpu as pltpu

_NEG = -1e30
_H = 32
_W = 32
_C = 128
_SP = _W + 3
_FLAT = 1360
_INT = 112
_M = _H * _SP
_CLEN = 1344


def _ring_rel():
    c = 3
    taps = [(dy, dx) for dy in range(7) for dx in range(7)]
    taps.sort(key=lambda t: (max(abs(t[0] - c), abs(t[1] - c)), t[0], t[1]))
    return [(dy - 3) * _SP + (dx - 3) for dy, dx in taps]


_REL = _ring_rel()

_ZA = [(0, 1), (2, 3), (4, 5), (6, 7), (8, None)]
_ZB = [(9 + 2 * k, 10 + 2 * k) for k in range(8)]
_ZC = [(25 + 2 * k, 26 + 2 * k) for k in range(12)]


def _interior_mask():
    f = np.arange(_FLAT)
    i, j = (f - _INT) // _SP, (f - _INT) % _SP
    m = (f >= _INT) & (i < _H) & (j < _W)
    return np.repeat(m[:, None].astype(np.float32), _C, axis=1)


_MASK = _interior_mask()
_MKEEP = _MASK[_INT:_INT + _M, :].astype(np.float32)
_MFILL = ((1.0 - _MKEEP) * _NEG).astype(np.float32)


def _module_kernel(x_ref, mask_ref,
                   wbn0_ref, sbn0_ref, w10_ref, s10_ref,
                   wa0_ref, wb0_ref, wc0_ref, s30_ref, s50_ref, s70_ref,
                   wbn1_ref, sbn1_ref, w11_ref, s11_ref,
                   wa1_ref, wb1_ref, wc1_ref, s31_ref, s51_ref, s71_ref,
                   mkeep_ref, mfill_ref,
                   o_ref, xflat, bnflat, bcopy, pbuf):
    f32 = jnp.float32
    bf16 = jnp.bfloat16
    mask = mask_ref[...]

    xflat[...] = jnp.full((_FLAT, _C), _NEG, bf16)

    def tap(t):
        a = _INT + _REL[t]
        s = a % 16
        if s == 0:
            return bnflat[a:a + _M, :]
        return bcopy[s - 1, a - s:a - s + _M, :]

    def ptap(i, j):
        a = tap(i)
        if j is None:
            return a
        b = tap(j)
        return a + jnp.concatenate([b[:, 64:128], b[:, 0:64]], axis=1)

    def one_layer(vwin, wbn_ref, sbn_ref, w1_ref, s1_ref,
                  wa_ref, wb_ref, wc_ref, s3_ref, s5_ref, s7_ref):
        xflat[_INT:_INT + _M, :] = (vwin * mkeep_ref[...]
                                    + mfill_ref[...])

        g0 = _INT - 2 * _SP - 2
        cl = 1260
        cm3 = jnp.maximum(
            jnp.maximum(xflat[g0 + 1:g0 + 1 + cl, :],
                        xflat[g0 + 2:g0 + 2 + cl, :]),
            xflat[g0 + 3:g0 + 3 + cl, :])
        cm5 = jnp.maximum(jnp.maximum(cm3, xflat[g0:g0 + cl, :]),
                          xflat[g0 + 4:g0 + 4 + cl, :])
        p3 = jnp.maximum(
            jnp.maximum(cm3[_SP:_SP + _M], cm3[2 * _SP:2 * _SP + _M]),
            cm3[3 * _SP:3 * _SP + _M])
        p5 = jnp.maximum(
            jnp.maximum(jnp.maximum(cm5[0:_M], cm5[_SP:_SP + _M]),
                        jnp.maximum(cm5[2 * _SP:2 * _SP + _M],
                                    cm5[3 * _SP:3 * _SP + _M])),
            cm5[4 * _SP:4 * _SP + _M])
        pbuf[...] = p3 + p5

        sbn = sbn_ref[...]
        bn = jnp.dot(xflat[...], wbn_ref[...], preferred_element_type=f32)
        bn = jnp.maximum(bn * sbn[0:1, :] + sbn[1:2, :], 0.0) * mask
        bnflat[...] = bn.astype(bf16)

        for s in range(1, 16):
            bcopy[s - 1, 0:_CLEN, :] = bnflat[s:s + _CLEN, :]

        s1 = s1_ref[...]
        cv1 = jnp.dot(tap(0), w1_ref[...], preferred_element_type=f32)
        o1 = jnp.maximum(cv1 * s1[0:1, :] + s1[1:2, :], 0.0)

        la = jnp.concatenate([ptap(i, j) for i, j in _ZA], axis=1)
        da = jnp.dot(la, wa_ref[...], preferred_element_type=f32)
        lb = jnp.concatenate([ptap(i, j) for i, j in _ZB], axis=1)
        db = jnp.dot(lb, wb_ref[...], preferred_element_type=f32)
        lc = jnp.concatenate([ptap(i, j) for i, j in _ZC], axis=1)
        dc = jnp.dot(lc, wc_ref[...], preferred_element_type=f32)

        s3 = s3_ref[...]
        s5 = s5_ref[...]
        s7 = s7_ref[...]
        o3 = jnp.maximum(da[:, 0:_C] * s3[0:1, :] + s3[1:2, :], 0.0)
        c5 = da[:, _C:2 * _C] + db[:, 0:_C]
        o5 = jnp.maximum(c5 * s5[0:1, :] + s5[1:2, :], 0.0)
        c7 = da[:, 2 * _C:3 * _C] + db[:, _C:2 * _C] + dc
        o7 = jnp.maximum(c7 * s7[0:1, :] + s7[1:2, :], 0.0)

        return o1 + o3 + o5 + o7 + pbuf[...].astype(f32)

    x0 = x_ref[0]
    v0 = jnp.concatenate(
        [x0.astype(bf16), jnp.zeros((_H, _SP - _W, _C), bf16)],
        axis=1).reshape(_M, _C)
    r0 = one_layer(v0, wbn0_ref, sbn0_ref, w10_ref, s10_ref,
                   wa0_ref, wb0_ref, wc0_ref, s30_ref, s50_ref, s70_ref)
    r1 = one_layer(r0.astype(bf16), wbn1_ref, sbn1_ref, w11_ref, s11_ref,
                   wa1_ref, wb1_ref, wc1_ref, s31_ref, s51_ref, s71_ref)
    o_ref[0] = r1.reshape(_H, _SP, _C)[:, 0:_W, :]


def _pack_rows(W, pairs):
    rows = []
    for i, j in pairs:
        rows.append(W[i * _C:i * _C + 64])
        rows.append(W[j * _C:j * _C + 64] if j is not None
                    else jnp.zeros((64, W.shape[1]), W.dtype))
    return jnp.concatenate(rows, axis=0)


def _prep_layer(wbn, sbn, w1, s1, w3, s3, w5, s5, w7, s7):
    bf16 = jnp.bfloat16
    k3 = w3.shape[0]
    wa = _pack_rows(jnp.concatenate([w3, w5[0:k3], w7[0:k3]], axis=1),
                    _ZA).astype(bf16)
    wb = _pack_rows(jnp.concatenate([w5, w7[0:w5.shape[0]]], axis=1),
                    _ZB).astype(bf16)
    wc = _pack_rows(w7, _ZC).astype(bf16)
    return (wbn.astype(bf16), sbn, w1.astype(bf16), s1,
            wa, wb, wc, s3, s5, s7)


def _const_spec(a):
    return pl.BlockSpec(a.shape, lambda b: (0, 0))


def kernel(x,
           L0_wbn, L0_sbn, L0_w1, L0_s1, L0_w3, L0_s3, L0_w5, L0_s5,
           L0_w7, L0_s7,
           L1_wbn, L1_sbn, L1_w1, L1_s1, L1_w3, L1_s3, L1_w5, L1_s5,
           L1_w7, L1_s7):
    N = x.shape[0]
    xh = jnp.transpose(x, (0, 2, 3, 1))
    consts = ((jnp.asarray(_MASK),)
              + _prep_layer(L0_wbn, L0_sbn, L0_w1, L0_s1, L0_w3, L0_s3,
                            L0_w5, L0_s5, L0_w7, L0_s7)
              + _prep_layer(L1_wbn, L1_sbn, L1_w1, L1_s1, L1_w3, L1_s3,
                            L1_w5, L1_s5, L1_w7, L1_s7)
              + (jnp.asarray(_MKEEP).astype(jnp.bfloat16),
                 jnp.asarray(_MFILL).astype(jnp.bfloat16)))
    in_specs = [pl.BlockSpec((1, _H, _W, _C), lambda b: (b, 0, 0, 0))]
    in_specs += [_const_spec(a) for a in consts]
    y = pl.pallas_call(
        _module_kernel,
        out_shape=jax.ShapeDtypeStruct((N, _H, _W, _C), jnp.float32),
        grid_spec=pltpu.PrefetchScalarGridSpec(
            num_scalar_prefetch=0, grid=(N,),
            in_specs=in_specs,
            out_specs=pl.BlockSpec((1, _H, _W, _C), lambda b: (b, 0, 0, 0)),
            scratch_shapes=[pltpu.VMEM((_FLAT, _C), jnp.bfloat16),
                            pltpu.VMEM((_FLAT, _C), jnp.bfloat16),
                            pltpu.VMEM((15, _CLEN, _C), jnp.bfloat16),
                            pltpu.VMEM((_M, _C), jnp.bfloat16)]),
        compiler_params=pltpu.CompilerParams(
            dimension_semantics=("parallel",),
            vmem_limit_bytes=60 * 1024 * 1024),
    )(xh, *consts)
    return jnp.transpose(y, (0, 3, 1, 2))
